# baseline (speedup 1.0000x reference)
import functools

import jax
import jax.numpy as jnp
from jax import lax
from jax.experimental import pallas as pl
from jax.experimental.pallas import tpu as pltpu
from jax.experimental.pallas import tpu_sc as plsc

N = 10000
E = 320000
H = 128
NPAD = 10240
NC = 2
NS = 16
RPT = NPAD // NS
CHUNK = 64
KG = 24
CH = 336
NG = CH // KG
EPT = CH * CHUNK
EP = NS * EPT

_mesh = plsc.VectorSubcoreMesh(core_axis_name="c", subcore_axis_name="s")


@functools.partial(
    pl.kernel,
    out_type=jax.ShapeDtypeStruct((2 * NPAD,), jnp.float32),
    mesh=_mesh,
    scratch_types=[
        pltpu.VMEM((EPT,), jnp.int32),
        pltpu.VMEM((NPAD,), jnp.float32),
        pltpu.VMEM_SHARED((NS, NPAD), jnp.float32),
        pltpu.VMEM((RPT,), jnp.float32),
        pltpu.VMEM((RPT,), jnp.float32),
    ],
    compiler_params=pltpu.CompilerParams(needs_layout_passes=False),
)
def _deg_kernel(sidx_hbm, deg_out, idx_v, hist_v, shared, tmp_v, acc_v):
    c = lax.axis_index("c")
    s = lax.axis_index("s")
    w = c * NS + s
    zero16 = jnp.zeros((16,), jnp.float32)
    ones16 = jnp.ones((16,), jnp.float32)

    def zbody(i, carry):
        hist_v[pl.ds(i * 16, 16)] = zero16
        return carry

    lax.fori_loop(0, NPAD // 16, zbody, 0)

    pltpu.sync_copy(sidx_hbm.at[w], idx_v)

    def hbody(i, carry):
        idx = idx_v[pl.ds(i * 16, 16)]
        plsc.addupdate_scatter(hist_v, [idx], ones16)
        return carry

    lax.fori_loop(0, EPT // 16, hbody, 0)

    pltpu.sync_copy(hist_v, shared.at[s])
    plsc.subcore_barrier()

    base = s * RPT
    for t in range(NS):
        pltpu.sync_copy(shared.at[t, pl.ds(base, RPT)], tmp_v)
        if t == 0:
            def cbody(k, carry):
                acc_v[pl.ds(k * 16, 16)] = tmp_v[pl.ds(k * 16, 16)]
                return carry
        else:
            def cbody(k, carry):
                acc_v[pl.ds(k * 16, 16)] = (
                    acc_v[pl.ds(k * 16, 16)] + tmp_v[pl.ds(k * 16, 16)]
                )
                return carry
        lax.fori_loop(0, RPT // 16, cbody, 0)

    pltpu.sync_copy(acc_v, deg_out.at[pl.ds(c * NPAD + base, RPT)])


@functools.partial(
    pl.kernel,
    out_type=jax.ShapeDtypeStruct((2 * NPAD, H), jnp.float32),
    mesh=_mesh,
    scratch_types=[
        pltpu.VMEM((2, KG, CHUNK), jnp.int32),
        pltpu.VMEM((2, KG, CHUNK), jnp.int32),
        pltpu.VMEM((4, CHUNK, H), jnp.float32),
        pltpu.VMEM_SHARED((NPAD, H), jnp.float32),
        pltpu.SemaphoreType.DMA,
        pltpu.SemaphoreType.DMA,
        pltpu.SemaphoreType.DMA,
        pltpu.SemaphoreType.DMA,
        pltpu.SemaphoreType.DMA,
    ],
)
def _agg_kernel(htil_hbm, gidx_hbm, sidx_hbm, zeros_hbm, acc_out,
                gi_v, si_v, rows_v, acc_sh, gsemA, gsemB, gsemC, isem,
                ssem):
    c = lax.axis_index("c")
    s = lax.axis_index("s")
    w = c * NS + s
    rbase = s * RPT

    pltpu.sync_copy(zeros_hbm.at[pl.ds(rbase, RPT)],
                    acc_sh.at[pl.ds(rbase, RPT)])
    plsc.subcore_barrier()

    gsems = (gsemA, gsemB, gsemC)
    pltpu.sync_copy(gidx_hbm.at[w, pl.ds(0, KG)], gi_v.at[0])
    pltpu.sync_copy(sidx_hbm.at[w, pl.ds(0, KG)], si_v.at[0])
    for k in range(3):
        pltpu.async_copy(htil_hbm.at[gi_v.at[0, k]], rows_v.at[k],
                         gsems[k % 3])

    def gbody(g, carry):
        gp = lax.rem(g, 2)
        has_next = g + 1 < NG

        @pl.when(has_next)
        def _prefetch():
            nxt = (g + 1) * KG
            pltpu.async_copy(gidx_hbm.at[w, pl.ds(nxt, KG)],
                             gi_v.at[1 - gp], isem)
            pltpu.async_copy(sidx_hbm.at[w, pl.ds(nxt, KG)],
                             si_v.at[1 - gp], isem)

        for j in range(KG):
            p = j % 4
            pltpu.make_async_copy(htil_hbm.at[gi_v.at[gp, j]],
                                  rows_v.at[p], gsems[j % 3]).wait()
            pltpu.async_copy(rows_v.at[p], acc_sh.at[si_v.at[gp, j]],
                             ssem, add=True)

            def _wait_scatter1():
                pltpu.make_async_copy(rows_v.at[(j + 3) % 4],
                                      acc_sh.at[si_v.at[gp, j]],
                                      ssem).wait()

            if j >= 1:
                _wait_scatter1()
            else:
                pl.when(g > 0)(_wait_scatter1)
            if j + 3 < KG:
                pltpu.async_copy(htil_hbm.at[gi_v.at[gp, j + 3]],
                                 rows_v.at[(j + 3) % 4], gsems[j % 3])
            else:
                if j + 3 == KG:
                    @pl.when(has_next)
                    def _idx_arrive():
                        pltpu.make_async_copy(
                            gidx_hbm.at[w, pl.ds(0, KG)], gi_v.at[1 - gp],
                            isem).wait()
                        pltpu.make_async_copy(
                            sidx_hbm.at[w, pl.ds(0, KG)], si_v.at[1 - gp],
                            isem).wait()

                @pl.when(has_next)
                def _next_group_gather():
                    pltpu.async_copy(
                        htil_hbm.at[gi_v.at[1 - gp, j + 3 - KG]],
                        rows_v.at[(j + 3) % 4], gsems[j % 3])
        return carry

    lax.fori_loop(0, NG, gbody, 0)
    pltpu.make_async_copy(rows_v.at[(CH - 1) % 4],
                          acc_sh.at[si_v.at[(NG - 1) % 2, KG - 1]],
                          ssem).wait()
    plsc.subcore_barrier()

    pltpu.sync_copy(acc_sh.at[pl.ds(rbase, RPT)],
                    acc_out.at[pl.ds(c * NPAD + rbase, RPT)])


BT = 1024
GRID = NPAD // BT


def _tc1_body(x_ref, degu_ref, degd_ref, wu_ref, wd_ref, out_ref):
    xb = x_ref[...]
    du = lax.rsqrt(degu_ref[...] + 1.0)
    dv = lax.rsqrt(degd_ref[...] + 1.0)
    hu = jnp.dot(xb, wu_ref[...], preferred_element_type=jnp.float32)
    hd = jnp.dot(xb, wd_ref[...], preferred_element_type=jnp.float32)
    out_ref[0] = hu * du
    out_ref[1] = hd * dv


def _tc1(xp, degu, degd, wu, wd):
    return pl.pallas_call(
        _tc1_body,
        grid=(GRID,),
        in_specs=[
            pl.BlockSpec((BT, H), lambda i: (i, 0)),
            pl.BlockSpec((BT, 1), lambda i: (i, 0)),
            pl.BlockSpec((BT, 1), lambda i: (i, 0)),
            pl.BlockSpec((H, H), lambda i: (0, 0)),
            pl.BlockSpec((H, H), lambda i: (0, 0)),
        ],
        out_specs=pl.BlockSpec((2, BT, H), lambda i: (0, i, 0)),
        out_shape=jax.ShapeDtypeStruct((2, NPAD, H), jnp.float32),
    )(xp, degu, degd, wu, wd)


def _pre_merge(x_ref, htil_ref, acc_ref, degu_ref, degd_ref,
               bu_ref, bd_ref, wlu_ref, wld_ref, bl_ref):
    xb = x_ref[...]
    du = lax.rsqrt(degu_ref[...] + 1.0)
    dv = lax.rsqrt(degd_ref[...] + 1.0)
    xu = xb + du * (acc_ref[0] + htil_ref[0]) + bu_ref[...]
    xd = xb + dv * (acc_ref[1] + htil_ref[1]) + bd_ref[...]
    ru = jnp.maximum(xu, 0.0)
    rd = jnp.maximum(xd, 0.0)
    y = (jnp.dot(ru, wlu_ref[...], preferred_element_type=jnp.float32)
         + jnp.dot(rd, wld_ref[...], preferred_element_type=jnp.float32)
         + bl_ref[...])
    return y


def _merge0_body(x_ref, htil_ref, acc_ref, degu_ref, degd_ref,
                 bu_ref, bd_ref, wlu_ref, wld_ref, bl_ref, g_ref, be_ref,
                 z_ref):
    y = _pre_merge(x_ref, htil_ref, acc_ref, degu_ref, degd_ref,
                   bu_ref, bd_ref, wlu_ref, wld_ref, bl_ref)
    ry = jnp.maximum(y, 0.0)
    mu = jnp.mean(ry, axis=-1, keepdims=True)
    var = jnp.mean((ry - mu) * (ry - mu), axis=-1, keepdims=True)
    z_ref[...] = g_ref[...] * (ry - mu) * lax.rsqrt(var + 1e-5) + be_ref[...]


def _merge0tc1_body(x_ref, htil_ref, acc_ref, degu_ref, degd_ref,
                    bu_ref, bd_ref, wlu_ref, wld_ref, bl_ref, g_ref, be_ref,
                    wu1_ref, wd1_ref, z_ref, out2_ref):
    y = _pre_merge(x_ref, htil_ref, acc_ref, degu_ref, degd_ref,
                   bu_ref, bd_ref, wlu_ref, wld_ref, bl_ref)
    ry = jnp.maximum(y, 0.0)
    mu = jnp.mean(ry, axis=-1, keepdims=True)
    var = jnp.mean((ry - mu) * (ry - mu), axis=-1, keepdims=True)
    z = g_ref[...] * (ry - mu) * lax.rsqrt(var + 1e-5) + be_ref[...]
    z_ref[...] = z
    du = lax.rsqrt(degu_ref[...] + 1.0)
    dv = lax.rsqrt(degd_ref[...] + 1.0)
    out2_ref[0] = jnp.dot(z, wu1_ref[...],
                          preferred_element_type=jnp.float32) * du
    out2_ref[1] = jnp.dot(z, wd1_ref[...],
                          preferred_element_type=jnp.float32) * dv


def _merge0tc1(xp, htil, acc, degu, degd, bu, bd, wlu, wld, bl, g, be,
               wu1, wd1):
    return pl.pallas_call(
        _merge0tc1_body,
        grid=(GRID,),
        in_specs=_MERGE_IN_SPECS + [
            pl.BlockSpec((1, H), lambda i: (0, 0)),
            pl.BlockSpec((1, H), lambda i: (0, 0)),
            pl.BlockSpec((H, H), lambda i: (0, 0)),
            pl.BlockSpec((H, H), lambda i: (0, 0)),
        ],
        out_specs=[
            pl.BlockSpec((BT, H), lambda i: (i, 0)),
            pl.BlockSpec((2, BT, H), lambda i: (0, i, 0)),
        ],
        out_shape=[
            jax.ShapeDtypeStruct((NPAD, H), jnp.float32),
            jax.ShapeDtypeStruct((2, NPAD, H), jnp.float32),
        ],
    )(xp, htil, acc, degu, degd, bu, bd, wlu, wld, bl, g, be, wu1, wd1)


def _merge1_body(x_ref, htil_ref, acc_ref, degu_ref, degd_ref,
                 bu_ref, bd_ref, wlu_ref, wld_ref, bl_ref,
                 emb_ref, out_ref):
    y = _pre_merge(x_ref, htil_ref, acc_ref, degu_ref, degd_ref,
                   bu_ref, bd_ref, wlu_ref, wld_ref, bl_ref)
    emb_ref[...] = y
    out_ref[...] = jnp.maximum(y, 0.0)


_MERGE_IN_SPECS = [
    pl.BlockSpec((BT, H), lambda i: (i, 0)),
    pl.BlockSpec((2, BT, H), lambda i: (0, i, 0)),
    pl.BlockSpec((2, BT, H), lambda i: (0, i, 0)),
    pl.BlockSpec((BT, 1), lambda i: (i, 0)),
    pl.BlockSpec((BT, 1), lambda i: (i, 0)),
    pl.BlockSpec((1, H), lambda i: (0, 0)),
    pl.BlockSpec((1, H), lambda i: (0, 0)),
    pl.BlockSpec((H, H), lambda i: (0, 0)),
    pl.BlockSpec((H, H), lambda i: (0, 0)),
    pl.BlockSpec((1, H), lambda i: (0, 0)),
]


def _merge0(xp, htil, acc, degu, degd, bu, bd, wlu, wld, bl, g, be):
    return pl.pallas_call(
        _merge0_body,
        grid=(GRID,),
        in_specs=_MERGE_IN_SPECS + [
            pl.BlockSpec((1, H), lambda i: (0, 0)),
            pl.BlockSpec((1, H), lambda i: (0, 0)),
        ],
        out_specs=pl.BlockSpec((BT, H), lambda i: (i, 0)),
        out_shape=jax.ShapeDtypeStruct((NPAD, H), jnp.float32),
    )(xp, htil, acc, degu, degd, bu, bd, wlu, wld, bl, g, be)


def _merge1(xp, htil, acc, degu, degd, bu, bd, wlu, wld, bl):
    return pl.pallas_call(
        _merge1_body,
        grid=(GRID,),
        in_specs=_MERGE_IN_SPECS,
        out_specs=[
            pl.BlockSpec((BT, H), lambda i: (i, 0)),
            pl.BlockSpec((BT, H), lambda i: (i, 0)),
        ],
        out_shape=[
            jax.ShapeDtypeStruct((NPAD, H), jnp.float32),
            jax.ShapeDtypeStruct((NPAD, H), jnp.float32),
        ],
    )(xp, htil, acc, degu, degd, bu, bd, wlu, wld, bl)


def kernel(x, edge_index, params):
    src = edge_index[0].astype(jnp.int32)
    dst = edge_index[1].astype(jnp.int32)
    pad = jnp.full((EP - E,), N, dtype=jnp.int32)
    srcp = jnp.concatenate([src, pad])
    dstp = jnp.concatenate([dst, pad])

    gidx = jnp.stack([srcp, dstp + NPAD]).reshape(NC * NS, CH, CHUNK)
    sidx = jnp.stack([dstp, srcp]).reshape(NC * NS, CH, CHUNK)
    sidx_flat = sidx.reshape(NC * NS, EPT)

    deg = _deg_kernel(sidx_flat)
    degu = deg[:NPAD].reshape(NPAD, 1)
    degd = deg[NPAD:].reshape(NPAD, 1)

    xp = jnp.pad(x, ((0, NPAD - N), (0, 0)))
    zeros = jnp.zeros((NPAD, H), jnp.float32)

    p = params
    htil0 = _tc1(xp, degu, degd, p['Wu0'], p['Wd0'])
    acc0 = _agg_kernel(htil0.reshape(2 * NPAD, H), gidx, sidx,
                       zeros).reshape(2, NPAD, H)
    wl0 = p['Wl0']
    z, htil1 = _merge0tc1(
        xp, htil0, acc0, degu, degd,
        p['bu0'].reshape(1, H), p['bd0'].reshape(1, H),
        wl0[:H], wl0[H:], p['bl0'].reshape(1, H),
        p['g0'].reshape(1, H), p['be0'].reshape(1, H),
        p['Wu1'], p['Wd1'])
    acc1 = _agg_kernel(htil1.reshape(2 * NPAD, H), gidx, sidx,
                       zeros).reshape(2, NPAD, H)
    wl1 = p['Wl1']
    emb, out = _merge1(
        z, htil1, acc1, degu, degd,
        p['bu1'].reshape(1, H), p['bd1'].reshape(1, H),
        wl1[:H], wl1[H:], p['bl1'].reshape(1, H))
    return emb[:N], out[:N]


# speedup vs baseline: 2.4921x; 2.4921x over previous
import functools

import jax
import jax.numpy as jnp
from jax import lax
from jax.experimental import pallas as pl
from jax.experimental.pallas import tpu as pltpu
from jax.experimental.pallas import tpu_sc as plsc

N = 10000
E = 320000
H = 128
NPAD = 10240
NC = 2
NS = 16
RPT = NPAD // NS
CHUNK = 64
KG = 32
CH = 320
NG = CH // KG
EPT = CH * CHUNK
EP = NS * EPT

_mesh = plsc.VectorSubcoreMesh(core_axis_name="c", subcore_axis_name="s")


@functools.partial(
    pl.kernel,
    out_type=jax.ShapeDtypeStruct((2 * NPAD,), jnp.float32),
    mesh=_mesh,
    scratch_types=[
        pltpu.VMEM((EPT,), jnp.int32),
        pltpu.VMEM((NPAD,), jnp.float32),
        pltpu.VMEM_SHARED((NS, NPAD), jnp.float32),
        pltpu.VMEM((RPT,), jnp.float32),
        pltpu.VMEM((RPT,), jnp.float32),
    ],
    compiler_params=pltpu.CompilerParams(needs_layout_passes=False),
)
def _deg_kernel(sidx_hbm, deg_out, idx_v, hist_v, shared, tmp_v, acc_v):
    c = lax.axis_index("c")
    s = lax.axis_index("s")
    w = c * NS + s
    zero16 = jnp.zeros((16,), jnp.float32)
    ones16 = jnp.ones((16,), jnp.float32)

    def zbody(i, carry):
        hist_v[pl.ds(i * 16, 16)] = zero16
        return carry

    lax.fori_loop(0, NPAD // 16, zbody, 0)

    pltpu.sync_copy(sidx_hbm.at[w], idx_v)

    def hbody(i, carry):
        idx = idx_v[pl.ds(i * 16, 16)]
        plsc.addupdate_scatter(hist_v, [idx], ones16)
        return carry

    lax.fori_loop(0, EPT // 16, hbody, 0)

    pltpu.sync_copy(hist_v, shared.at[s])
    plsc.subcore_barrier()

    base = s * RPT
    for t in range(NS):
        pltpu.sync_copy(shared.at[t, pl.ds(base, RPT)], tmp_v)
        if t == 0:
            def cbody(k, carry):
                acc_v[pl.ds(k * 16, 16)] = tmp_v[pl.ds(k * 16, 16)]
                return carry
        else:
            def cbody(k, carry):
                acc_v[pl.ds(k * 16, 16)] = (
                    acc_v[pl.ds(k * 16, 16)] + tmp_v[pl.ds(k * 16, 16)]
                )
                return carry
        lax.fori_loop(0, RPT // 16, cbody, 0)

    pltpu.sync_copy(acc_v, deg_out.at[pl.ds(c * NPAD + base, RPT)])


@functools.partial(
    pl.kernel,
    out_type=jax.ShapeDtypeStruct((2 * NPAD, H), jnp.float32),
    mesh=_mesh,
    scratch_types=[
        pltpu.VMEM((2, KG, CHUNK), jnp.int32),
        pltpu.VMEM((2, KG, CHUNK), jnp.int32),
        pltpu.VMEM((4, CHUNK, H), jnp.float32),
        pltpu.VMEM_SHARED((NPAD, H), jnp.float32),
        pltpu.SemaphoreType.DMA,
        pltpu.SemaphoreType.DMA,
        pltpu.SemaphoreType.DMA,
        pltpu.SemaphoreType.DMA,
    ],
)
def _agg_kernel(htil_hbm, gidx_hbm, sidx_hbm, zeros_hbm, acc_out,
                gi_v, si_v, rows_v, acc_sh, gsemA, gsemB, isem, ssem):
    c = lax.axis_index("c")
    s = lax.axis_index("s")
    w = c * NS + s
    rbase = s * RPT

    pltpu.sync_copy(zeros_hbm.at[pl.ds(rbase, RPT)],
                    acc_sh.at[pl.ds(rbase, RPT)])
    plsc.subcore_barrier()

    gsems = (gsemA, gsemB)
    pltpu.sync_copy(gidx_hbm.at[w, pl.ds(0, KG)], gi_v.at[0])
    pltpu.sync_copy(sidx_hbm.at[w, pl.ds(0, KG)], si_v.at[0])
    for k in range(2):
        pltpu.async_copy(htil_hbm.at[gi_v.at[0, k]], rows_v.at[k],
                         gsems[k % 2])

    def gbody(g, carry):
        gp = lax.rem(g, 2)
        has_next = g + 1 < NG

        @pl.when(has_next)
        def _prefetch():
            nxt = (g + 1) * KG
            pltpu.async_copy(gidx_hbm.at[w, pl.ds(nxt, KG)],
                             gi_v.at[1 - gp], isem)
            pltpu.async_copy(sidx_hbm.at[w, pl.ds(nxt, KG)],
                             si_v.at[1 - gp], isem)

        for j in range(KG):
            p = j % 4
            pltpu.make_async_copy(htil_hbm.at[gi_v.at[gp, j]],
                                  rows_v.at[p], gsems[j % 2]).wait()
            pltpu.async_copy(rows_v.at[p], acc_sh.at[si_v.at[gp, j]],
                             ssem, add=True)

            def _wait_scatter2():
                pltpu.make_async_copy(rows_v.at[(j + 2) % 4],
                                      acc_sh.at[si_v.at[gp, j]],
                                      ssem).wait()

            if j >= 2:
                _wait_scatter2()
            else:
                pl.when(g > 0)(_wait_scatter2)
            if j + 2 < KG:
                pltpu.async_copy(htil_hbm.at[gi_v.at[gp, j + 2]],
                                 rows_v.at[(j + 2) % 4], gsems[j % 2])
            else:
                if j + 2 == KG:
                    @pl.when(has_next)
                    def _idx_arrive():
                        pltpu.make_async_copy(
                            gidx_hbm.at[w, pl.ds(0, KG)], gi_v.at[1 - gp],
                            isem).wait()
                        pltpu.make_async_copy(
                            sidx_hbm.at[w, pl.ds(0, KG)], si_v.at[1 - gp],
                            isem).wait()

                @pl.when(has_next)
                def _next_group_gather():
                    pltpu.async_copy(
                        htil_hbm.at[gi_v.at[1 - gp, j + 2 - KG]],
                        rows_v.at[(j + 2) % 4], gsems[j % 2])
        return carry

    lax.fori_loop(0, NG, gbody, 0)
    for k in range(2):
        pltpu.make_async_copy(rows_v.at[(CH - 2 + k) % 4],
                              acc_sh.at[si_v.at[(NG - 1) % 2, KG - 2 + k]],
                              ssem).wait()
    plsc.subcore_barrier()

    pltpu.sync_copy(acc_sh.at[pl.ds(rbase, RPT)],
                    acc_out.at[pl.ds(c * NPAD + rbase, RPT)])


BT = 1024
GRID = NPAD // BT


def _tc1_body(x_ref, degu_ref, degd_ref, wu_ref, wd_ref, out_ref):
    xb = x_ref[...]
    du = lax.rsqrt(degu_ref[...] + 1.0)
    dv = lax.rsqrt(degd_ref[...] + 1.0)
    hu = jnp.dot(xb, wu_ref[...], preferred_element_type=jnp.float32)
    hd = jnp.dot(xb, wd_ref[...], preferred_element_type=jnp.float32)
    out_ref[0] = hu * du
    out_ref[1] = hd * dv


def _tc1(xp, degu, degd, wu, wd):
    return pl.pallas_call(
        _tc1_body,
        grid=(GRID,),
        in_specs=[
            pl.BlockSpec((BT, H), lambda i: (i, 0)),
            pl.BlockSpec((BT, 1), lambda i: (i, 0)),
            pl.BlockSpec((BT, 1), lambda i: (i, 0)),
            pl.BlockSpec((H, H), lambda i: (0, 0)),
            pl.BlockSpec((H, H), lambda i: (0, 0)),
        ],
        out_specs=pl.BlockSpec((2, BT, H), lambda i: (0, i, 0)),
        out_shape=jax.ShapeDtypeStruct((2, NPAD, H), jnp.float32),
    )(xp, degu, degd, wu, wd)


def _pre_merge(x_ref, htil_ref, acc_ref, degu_ref, degd_ref,
               bu_ref, bd_ref, wlu_ref, wld_ref, bl_ref):
    xb = x_ref[...]
    du = lax.rsqrt(degu_ref[...] + 1.0)
    dv = lax.rsqrt(degd_ref[...] + 1.0)
    xu = xb + du * (acc_ref[0] + htil_ref[0]) + bu_ref[...]
    xd = xb + dv * (acc_ref[1] + htil_ref[1]) + bd_ref[...]
    ru = jnp.maximum(xu, 0.0)
    rd = jnp.maximum(xd, 0.0)
    y = (jnp.dot(ru, wlu_ref[...], preferred_element_type=jnp.float32)
         + jnp.dot(rd, wld_ref[...], preferred_element_type=jnp.float32)
         + bl_ref[...])
    return y


def _merge0_body(x_ref, htil_ref, acc_ref, degu_ref, degd_ref,
                 bu_ref, bd_ref, wlu_ref, wld_ref, bl_ref, g_ref, be_ref,
                 z_ref):
    y = _pre_merge(x_ref, htil_ref, acc_ref, degu_ref, degd_ref,
                   bu_ref, bd_ref, wlu_ref, wld_ref, bl_ref)
    ry = jnp.maximum(y, 0.0)
    mu = jnp.mean(ry, axis=-1, keepdims=True)
    var = jnp.mean((ry - mu) * (ry - mu), axis=-1, keepdims=True)
    z_ref[...] = g_ref[...] * (ry - mu) * lax.rsqrt(var + 1e-5) + be_ref[...]


def _merge0tc1_body(x_ref, htil_ref, acc_ref, degu_ref, degd_ref,
                    bu_ref, bd_ref, wlu_ref, wld_ref, bl_ref, g_ref, be_ref,
                    wu1_ref, wd1_ref, z_ref, out2_ref):
    y = _pre_merge(x_ref, htil_ref, acc_ref, degu_ref, degd_ref,
                   bu_ref, bd_ref, wlu_ref, wld_ref, bl_ref)
    ry = jnp.maximum(y, 0.0)
    mu = jnp.mean(ry, axis=-1, keepdims=True)
    var = jnp.mean((ry - mu) * (ry - mu), axis=-1, keepdims=True)
    z = g_ref[...] * (ry - mu) * lax.rsqrt(var + 1e-5) + be_ref[...]
    z_ref[...] = z
    du = lax.rsqrt(degu_ref[...] + 1.0)
    dv = lax.rsqrt(degd_ref[...] + 1.0)
    out2_ref[0] = jnp.dot(z, wu1_ref[...],
                          preferred_element_type=jnp.float32) * du
    out2_ref[1] = jnp.dot(z, wd1_ref[...],
                          preferred_element_type=jnp.float32) * dv


def _merge0tc1(xp, htil, acc, degu, degd, bu, bd, wlu, wld, bl, g, be,
               wu1, wd1):
    return pl.pallas_call(
        _merge0tc1_body,
        grid=(GRID,),
        in_specs=_MERGE_IN_SPECS + [
            pl.BlockSpec((1, H), lambda i: (0, 0)),
            pl.BlockSpec((1, H), lambda i: (0, 0)),
            pl.BlockSpec((H, H), lambda i: (0, 0)),
            pl.BlockSpec((H, H), lambda i: (0, 0)),
        ],
        out_specs=[
            pl.BlockSpec((BT, H), lambda i: (i, 0)),
            pl.BlockSpec((2, BT, H), lambda i: (0, i, 0)),
        ],
        out_shape=[
            jax.ShapeDtypeStruct((NPAD, H), jnp.float32),
            jax.ShapeDtypeStruct((2, NPAD, H), jnp.float32),
        ],
    )(xp, htil, acc, degu, degd, bu, bd, wlu, wld, bl, g, be, wu1, wd1)


def _merge1_body(x_ref, htil_ref, acc_ref, degu_ref, degd_ref,
                 bu_ref, bd_ref, wlu_ref, wld_ref, bl_ref,
                 emb_ref, out_ref):
    y = _pre_merge(x_ref, htil_ref, acc_ref, degu_ref, degd_ref,
                   bu_ref, bd_ref, wlu_ref, wld_ref, bl_ref)
    emb_ref[...] = y
    out_ref[...] = jnp.maximum(y, 0.0)


_MERGE_IN_SPECS = [
    pl.BlockSpec((BT, H), lambda i: (i, 0)),
    pl.BlockSpec((2, BT, H), lambda i: (0, i, 0)),
    pl.BlockSpec((2, BT, H), lambda i: (0, i, 0)),
    pl.BlockSpec((BT, 1), lambda i: (i, 0)),
    pl.BlockSpec((BT, 1), lambda i: (i, 0)),
    pl.BlockSpec((1, H), lambda i: (0, 0)),
    pl.BlockSpec((1, H), lambda i: (0, 0)),
    pl.BlockSpec((H, H), lambda i: (0, 0)),
    pl.BlockSpec((H, H), lambda i: (0, 0)),
    pl.BlockSpec((1, H), lambda i: (0, 0)),
]


def _merge0(xp, htil, acc, degu, degd, bu, bd, wlu, wld, bl, g, be):
    return pl.pallas_call(
        _merge0_body,
        grid=(GRID,),
        in_specs=_MERGE_IN_SPECS + [
            pl.BlockSpec((1, H), lambda i: (0, 0)),
            pl.BlockSpec((1, H), lambda i: (0, 0)),
        ],
        out_specs=pl.BlockSpec((BT, H), lambda i: (i, 0)),
        out_shape=jax.ShapeDtypeStruct((NPAD, H), jnp.float32),
    )(xp, htil, acc, degu, degd, bu, bd, wlu, wld, bl, g, be)


def _merge1(xp, htil, acc, degu, degd, bu, bd, wlu, wld, bl):
    return pl.pallas_call(
        _merge1_body,
        grid=(GRID,),
        in_specs=_MERGE_IN_SPECS,
        out_specs=[
            pl.BlockSpec((BT, H), lambda i: (i, 0)),
            pl.BlockSpec((BT, H), lambda i: (i, 0)),
        ],
        out_shape=[
            jax.ShapeDtypeStruct((NPAD, H), jnp.float32),
            jax.ShapeDtypeStruct((NPAD, H), jnp.float32),
        ],
    )(xp, htil, acc, degu, degd, bu, bd, wlu, wld, bl)


def kernel(x, edge_index, params):
    src = edge_index[0].astype(jnp.int32)
    dst = edge_index[1].astype(jnp.int32)
    pad = jnp.full((EP - E,), N, dtype=jnp.int32)
    srcp = jnp.concatenate([src, pad])
    dstp = jnp.concatenate([dst, pad])

    gidx = jnp.stack([srcp, dstp + NPAD]).reshape(NC * NS, CH, CHUNK)
    sidx = jnp.stack([dstp, srcp]).reshape(NC * NS, CH, CHUNK)
    sidx_flat = sidx.reshape(NC * NS, EPT)

    deg = _deg_kernel(sidx_flat)
    degu = deg[:NPAD].reshape(NPAD, 1)
    degd = deg[NPAD:].reshape(NPAD, 1)

    xp = jnp.pad(x, ((0, NPAD - N), (0, 0)))
    zeros = jnp.zeros((NPAD, H), jnp.float32)

    p = params
    htil0 = _tc1(xp, degu, degd, p['Wu0'], p['Wd0'])
    acc0 = _agg_kernel(htil0.reshape(2 * NPAD, H), gidx, sidx,
                       zeros).reshape(2, NPAD, H)
    wl0 = p['Wl0']
    z, htil1 = _merge0tc1(
        xp, htil0, acc0, degu, degd,
        p['bu0'].reshape(1, H), p['bd0'].reshape(1, H),
        wl0[:H], wl0[H:], p['bl0'].reshape(1, H),
        p['g0'].reshape(1, H), p['be0'].reshape(1, H),
        p['Wu1'], p['Wd1'])
    acc1 = _agg_kernel(htil1.reshape(2 * NPAD, H), gidx, sidx,
                       zeros).reshape(2, NPAD, H)
    wl1 = p['Wl1']
    emb, out = _merge1(
        z, htil1, acc1, degu, degd,
        p['bu1'].reshape(1, H), p['bd1'].reshape(1, H),
        wl1[:H], wl1[H:], p['bl1'].reshape(1, H))
    return emb[:N], out[:N]


# speedup vs baseline: 2.4923x; 1.0001x over previous
import functools

import jax
import jax.numpy as jnp
from jax import lax
from jax.experimental import pallas as pl
from jax.experimental.pallas import tpu as pltpu
from jax.experimental.pallas import tpu_sc as plsc

N = 10000
E = 320000
H = 128
NPAD = 10240
NC = 2
NS = 16
RPT = NPAD // NS
CHUNK = 64
KG = 32
CH = 320
NG = CH // KG
EPT = CH * CHUNK
EP = NS * EPT

_mesh = plsc.VectorSubcoreMesh(core_axis_name="c", subcore_axis_name="s")


@functools.partial(
    pl.kernel,
    out_type=jax.ShapeDtypeStruct((2 * NPAD,), jnp.float32),
    mesh=_mesh,
    scratch_types=[
        pltpu.VMEM((EPT,), jnp.int32),
        pltpu.VMEM((NPAD,), jnp.float32),
        pltpu.VMEM_SHARED((NS, NPAD), jnp.float32),
        pltpu.VMEM((RPT,), jnp.float32),
        pltpu.VMEM((RPT,), jnp.float32),
    ],
    compiler_params=pltpu.CompilerParams(needs_layout_passes=False),
)
def _deg_kernel(sidx_hbm, deg_out, idx_v, hist_v, shared, tmp_v, acc_v):
    c = lax.axis_index("c")
    s = lax.axis_index("s")
    w = c * NS + s
    zero16 = jnp.zeros((16,), jnp.float32)
    ones16 = jnp.ones((16,), jnp.float32)

    def zbody(i, carry):
        hist_v[pl.ds(i * 16, 16)] = zero16
        return carry

    lax.fori_loop(0, NPAD // 16, zbody, 0)

    pltpu.sync_copy(sidx_hbm.at[w], idx_v)

    def hbody(i, carry):
        idx = idx_v[pl.ds(i * 16, 16)]
        plsc.addupdate_scatter(hist_v, [idx], ones16)
        return carry

    lax.fori_loop(0, EPT // 16, hbody, 0)

    pltpu.sync_copy(hist_v, shared.at[s])
    plsc.subcore_barrier()

    base = s * RPT
    for t in range(NS):
        pltpu.sync_copy(shared.at[t, pl.ds(base, RPT)], tmp_v)
        if t == 0:
            def cbody(k, carry):
                acc_v[pl.ds(k * 16, 16)] = tmp_v[pl.ds(k * 16, 16)]
                return carry
        else:
            def cbody(k, carry):
                acc_v[pl.ds(k * 16, 16)] = (
                    acc_v[pl.ds(k * 16, 16)] + tmp_v[pl.ds(k * 16, 16)]
                )
                return carry
        lax.fori_loop(0, RPT // 16, cbody, 0)

    pltpu.sync_copy(acc_v, deg_out.at[pl.ds(c * NPAD + base, RPT)])


@functools.partial(
    pl.kernel,
    out_type=jax.ShapeDtypeStruct((2 * NPAD, H), jnp.float32),
    mesh=_mesh,
    scratch_types=[
        pltpu.VMEM((2, KG, CHUNK), jnp.int32),
        pltpu.VMEM((2, KG, CHUNK), jnp.int32),
        pltpu.VMEM((4, CHUNK, H), jnp.float32),
        pltpu.VMEM_SHARED((NPAD, H), jnp.float32),
        pltpu.SemaphoreType.DMA,
        pltpu.SemaphoreType.DMA,
        pltpu.SemaphoreType.DMA,
        pltpu.SemaphoreType.DMA,
    ],
)
def _agg_kernel(htil_hbm, gidx_hbm, sidx_hbm, zeros_hbm, acc_out,
                gi_v, si_v, rows_v, acc_sh, gsemA, gsemB, isem, ssem):
    c = lax.axis_index("c")
    s = lax.axis_index("s")
    w = c * NS + s
    rbase = s * RPT

    pltpu.sync_copy(zeros_hbm.at[pl.ds(rbase, RPT)],
                    acc_sh.at[pl.ds(rbase, RPT)])
    plsc.subcore_barrier()

    gsems = (gsemA, gsemB)
    pltpu.sync_copy(gidx_hbm.at[w, pl.ds(0, KG)], gi_v.at[0])
    pltpu.sync_copy(sidx_hbm.at[w, pl.ds(0, KG)], si_v.at[0])
    for k in range(2):
        pltpu.async_copy(htil_hbm.at[gi_v.at[0, k]], rows_v.at[k],
                         gsems[k % 2])

    def gbody(g, carry):
        gp = lax.rem(g, 2)
        has_next = g + 1 < NG

        @pl.when(has_next)
        def _prefetch():
            nxt = (g + 1) * KG
            pltpu.async_copy(gidx_hbm.at[w, pl.ds(nxt, KG)],
                             gi_v.at[1 - gp], isem)
            pltpu.async_copy(sidx_hbm.at[w, pl.ds(nxt, KG)],
                             si_v.at[1 - gp], isem)

        for j in range(KG):
            p = j % 4
            pltpu.make_async_copy(htil_hbm.at[gi_v.at[gp, j]],
                                  rows_v.at[p], gsems[j % 2]).wait()
            pltpu.async_copy(rows_v.at[p], acc_sh.at[si_v.at[gp, j]],
                             ssem, add=True)

            def _wait_scatter2():
                pltpu.make_async_copy(rows_v.at[(j + 2) % 4],
                                      acc_sh.at[si_v.at[gp, j]],
                                      ssem).wait()

            if j >= 2:
                _wait_scatter2()
            else:
                pl.when(g > 0)(_wait_scatter2)
            if j + 2 < KG:
                pltpu.async_copy(htil_hbm.at[gi_v.at[gp, j + 2]],
                                 rows_v.at[(j + 2) % 4], gsems[j % 2])
            else:
                if j + 2 == KG:
                    @pl.when(has_next)
                    def _idx_arrive():
                        pltpu.make_async_copy(
                            gidx_hbm.at[w, pl.ds(0, KG)], gi_v.at[1 - gp],
                            isem).wait()
                        pltpu.make_async_copy(
                            sidx_hbm.at[w, pl.ds(0, KG)], si_v.at[1 - gp],
                            isem).wait()

                @pl.when(has_next)
                def _next_group_gather():
                    pltpu.async_copy(
                        htil_hbm.at[gi_v.at[1 - gp, j + 2 - KG]],
                        rows_v.at[(j + 2) % 4], gsems[j % 2])
        return carry

    lax.fori_loop(0, NG, gbody, 0)
    for k in range(2):
        pltpu.make_async_copy(rows_v.at[(CH - 2 + k) % 4],
                              acc_sh.at[si_v.at[(NG - 1) % 2, KG - 2 + k]],
                              ssem).wait()
    plsc.subcore_barrier()

    pltpu.sync_copy(acc_sh.at[pl.ds(rbase, RPT)],
                    acc_out.at[pl.ds(c * NPAD + rbase, RPT)])


BT = 1024
GRID = NPAD // BT


def _tc1_body(x_ref, degu_ref, degd_ref, wu_ref, wd_ref, out_ref):
    xb = x_ref[...]
    du = lax.rsqrt(degu_ref[...] + 1.0)
    dv = lax.rsqrt(degd_ref[...] + 1.0)
    hu = jnp.dot(xb, wu_ref[...], preferred_element_type=jnp.float32)
    hd = jnp.dot(xb, wd_ref[...], preferred_element_type=jnp.float32)
    out_ref[0] = hu * du
    out_ref[1] = hd * dv


def _tc1(xp, degu, degd, wu, wd):
    return pl.pallas_call(
        _tc1_body,
        grid=(GRID,),
        in_specs=[
            pl.BlockSpec((BT, H), lambda i: (i, 0)),
            pl.BlockSpec((BT, 1), lambda i: (i, 0)),
            pl.BlockSpec((BT, 1), lambda i: (i, 0)),
            pl.BlockSpec((H, H), lambda i: (0, 0)),
            pl.BlockSpec((H, H), lambda i: (0, 0)),
        ],
        out_specs=pl.BlockSpec((2, BT, H), lambda i: (0, i, 0)),
        out_shape=jax.ShapeDtypeStruct((2, NPAD, H), jnp.float32),
    )(xp, degu, degd, wu, wd)


def _pre_merge(x_ref, htil_ref, acc_ref, degu_ref, degd_ref,
               bu_ref, bd_ref, wlu_ref, wld_ref, bl_ref):
    xb = x_ref[...]
    du = lax.rsqrt(degu_ref[...] + 1.0)
    dv = lax.rsqrt(degd_ref[...] + 1.0)
    xu = xb + du * (acc_ref[0] + htil_ref[0]) + bu_ref[...]
    xd = xb + dv * (acc_ref[1] + htil_ref[1]) + bd_ref[...]
    ru = jnp.maximum(xu, 0.0)
    rd = jnp.maximum(xd, 0.0)
    y = (jnp.dot(ru, wlu_ref[...], preferred_element_type=jnp.float32)
         + jnp.dot(rd, wld_ref[...], preferred_element_type=jnp.float32)
         + bl_ref[...])
    return y


def _merge0tc1_body(x_ref, htil_ref, acc_ref, degu_ref, degd_ref,
                    bu_ref, bd_ref, wlu_ref, wld_ref, bl_ref, g_ref, be_ref,
                    wu1_ref, wd1_ref, z_ref, out2_ref):
    y = _pre_merge(x_ref, htil_ref, acc_ref, degu_ref, degd_ref,
                   bu_ref, bd_ref, wlu_ref, wld_ref, bl_ref)
    ry = jnp.maximum(y, 0.0)
    mu = jnp.mean(ry, axis=-1, keepdims=True)
    var = jnp.mean((ry - mu) * (ry - mu), axis=-1, keepdims=True)
    z = g_ref[...] * (ry - mu) * lax.rsqrt(var + 1e-5) + be_ref[...]
    z_ref[...] = z
    du = lax.rsqrt(degu_ref[...] + 1.0)
    dv = lax.rsqrt(degd_ref[...] + 1.0)
    out2_ref[0] = jnp.dot(z, wu1_ref[...],
                          preferred_element_type=jnp.float32) * du
    out2_ref[1] = jnp.dot(z, wd1_ref[...],
                          preferred_element_type=jnp.float32) * dv


def _merge0tc1(xp, htil, acc, degu, degd, bu, bd, wlu, wld, bl, g, be,
               wu1, wd1):
    return pl.pallas_call(
        _merge0tc1_body,
        grid=(GRID,),
        in_specs=_MERGE_IN_SPECS + [
            pl.BlockSpec((1, H), lambda i: (0, 0)),
            pl.BlockSpec((1, H), lambda i: (0, 0)),
            pl.BlockSpec((H, H), lambda i: (0, 0)),
            pl.BlockSpec((H, H), lambda i: (0, 0)),
        ],
        out_specs=[
            pl.BlockSpec((BT, H), lambda i: (i, 0)),
            pl.BlockSpec((2, BT, H), lambda i: (0, i, 0)),
        ],
        out_shape=[
            jax.ShapeDtypeStruct((NPAD, H), jnp.float32),
            jax.ShapeDtypeStruct((2, NPAD, H), jnp.float32),
        ],
    )(xp, htil, acc, degu, degd, bu, bd, wlu, wld, bl, g, be, wu1, wd1)


def _merge1_body(x_ref, htil_ref, acc_ref, degu_ref, degd_ref,
                 bu_ref, bd_ref, wlu_ref, wld_ref, bl_ref,
                 emb_ref, out_ref):
    y = _pre_merge(x_ref, htil_ref, acc_ref, degu_ref, degd_ref,
                   bu_ref, bd_ref, wlu_ref, wld_ref, bl_ref)
    emb_ref[...] = y
    out_ref[...] = jnp.maximum(y, 0.0)


_MERGE_IN_SPECS = [
    pl.BlockSpec((BT, H), lambda i: (i, 0)),
    pl.BlockSpec((2, BT, H), lambda i: (0, i, 0)),
    pl.BlockSpec((2, BT, H), lambda i: (0, i, 0)),
    pl.BlockSpec((BT, 1), lambda i: (i, 0)),
    pl.BlockSpec((BT, 1), lambda i: (i, 0)),
    pl.BlockSpec((1, H), lambda i: (0, 0)),
    pl.BlockSpec((1, H), lambda i: (0, 0)),
    pl.BlockSpec((H, H), lambda i: (0, 0)),
    pl.BlockSpec((H, H), lambda i: (0, 0)),
    pl.BlockSpec((1, H), lambda i: (0, 0)),
]


def _merge1(xp, htil, acc, degu, degd, bu, bd, wlu, wld, bl):
    return pl.pallas_call(
        _merge1_body,
        grid=(GRID,),
        in_specs=_MERGE_IN_SPECS,
        out_specs=[
            pl.BlockSpec((BT, H), lambda i: (i, 0)),
            pl.BlockSpec((BT, H), lambda i: (i, 0)),
        ],
        out_shape=[
            jax.ShapeDtypeStruct((NPAD, H), jnp.float32),
            jax.ShapeDtypeStruct((NPAD, H), jnp.float32),
        ],
    )(xp, htil, acc, degu, degd, bu, bd, wlu, wld, bl)


def kernel(x, edge_index, params):
    src = edge_index[0].astype(jnp.int32)
    dst = edge_index[1].astype(jnp.int32)
    pad = jnp.full((EP - E,), N, dtype=jnp.int32)
    srcp = jnp.concatenate([src, pad])
    dstp = jnp.concatenate([dst, pad])

    gidx = jnp.stack([srcp, dstp + NPAD]).reshape(NC * NS, CH, CHUNK)
    sidx = jnp.stack([dstp, srcp]).reshape(NC * NS, CH, CHUNK)
    sidx_flat = sidx.reshape(NC * NS, EPT)

    deg = _deg_kernel(sidx_flat)
    degu = deg[:NPAD].reshape(NPAD, 1)
    degd = deg[NPAD:].reshape(NPAD, 1)

    xp = jnp.pad(x, ((0, NPAD - N), (0, 0)))
    zeros = jnp.zeros((NPAD, H), jnp.float32)

    p = params
    htil0 = _tc1(xp, degu, degd, p['Wu0'], p['Wd0'])
    acc0 = _agg_kernel(htil0.reshape(2 * NPAD, H), gidx, sidx,
                       zeros).reshape(2, NPAD, H)
    wl0 = p['Wl0']
    z, htil1 = _merge0tc1(
        xp, htil0, acc0, degu, degd,
        p['bu0'].reshape(1, H), p['bd0'].reshape(1, H),
        wl0[:H], wl0[H:], p['bl0'].reshape(1, H),
        p['g0'].reshape(1, H), p['be0'].reshape(1, H),
        p['Wu1'], p['Wd1'])
    acc1 = _agg_kernel(htil1.reshape(2 * NPAD, H), gidx, sidx,
                       zeros).reshape(2, NPAD, H)
    wl1 = p['Wl1']
    emb, out = _merge1(
        z, htil1, acc1, degu, degd,
        p['bu1'].reshape(1, H), p['bd1'].reshape(1, H),
        wl1[:H], wl1[H:], p['bl1'].reshape(1, H))
    return emb[:N], out[:N]
